# baseline (speedup 1.0000x reference)
import functools

import jax
import jax.numpy as jnp
from jax import lax
from jax.experimental import pallas as pl
from jax.experimental.pallas import tpu as pltpu
from jax.experimental.pallas import tpu_sc as plsc

_GAMMA = 0.9
_NPOS = 1024
_NNEG = 8192
_NTOT = 9216
_CHUNK = 1024


_info = plsc.get_sparse_core_info()
_NC = _info.num_cores
_NS = _info.num_subcores
_NW = _NC * _NS
_BPW = _NPOS // _NW


@functools.partial(
    pl.kernel,
    out_type=[
        jax.ShapeDtypeStruct((_NPOS, 1), jnp.float32),
        jax.ShapeDtypeStruct((_NPOS, 1), jnp.float32),
    ],
    mesh=plsc.VectorSubcoreMesh(core_axis_name="c", subcore_axis_name="s"),
    scratch_types=[
        pltpu.VMEM((_BPW,), jnp.int32),
        pltpu.VMEM((_BPW, 1), jnp.float32),
        pltpu.VMEM((_BPW, 1), jnp.float32),
        pltpu.SemaphoreType.DMA,
        pltpu.SemaphoreType.DMA,
    ],
    compiler_params=pltpu.CompilerParams(use_tc_tiling_on_sc=False),
)
def _sc_gather(idx_hbm, ua_hbm, up_hbm, oa_hbm, op_hbm,
               idx_v, a_v, p_v, sem_a, sem_p):
    wid = lax.axis_index("s") * _NC + lax.axis_index("c")
    base = wid * _BPW
    pltpu.sync_copy(idx_hbm.at[pl.ds(base, _BPW)], idx_v)
    cp_a = pltpu.async_copy(ua_hbm.at[idx_v], a_v, sem_a)
    cp_p = pltpu.async_copy(up_hbm.at[idx_v], p_v, sem_p)
    cp_a.wait()
    cp_p.wait()
    pltpu.sync_copy(a_v, oa_hbm.at[pl.ds(base, _BPW)])
    pltpu.sync_copy(p_v, op_hbm.at[pl.ds(base, _BPW)])


def _dense_body(fp_ref, fn_ref, idx_ref, out_ref):
    fp_row = fp_ref[...].reshape(1, _NPOS)
    fp_col = jnp.transpose(fp_row)
    a1 = 1.0 - fp_col

    hb = jnp.maximum(a1 + fp_row, 0.0)
    t_col = jnp.sum(hb * hb, axis=1, keepdims=True)
    s_col = t_col
    for c in range(_NNEG // _CHUNK):
        v = fn_ref[...].reshape(1, _NNEG)[:, c * _CHUNK:(c + 1) * _CHUNK]
        hb = jnp.maximum(a1 + v, 0.0)
        s_col = s_col + jnp.sum(hb * hb, axis=1, keepdims=True)

    ir = idx_ref[...].reshape(1, _NPOS)
    ic = jnp.transpose(ir)
    eq = ic == ir
    cid0 = lax.broadcasted_iota(jnp.int32, (_NPOS, _NPOS), 0)
    wmat = jnp.where(eq, cid0, -1)
    w_row = jnp.max(wmat, axis=0, keepdims=True)
    onehot = (wmat == w_row).astype(jnp.float32)

    sw_row = jnp.sum(onehot * s_col, axis=0, keepdims=True)
    tw_row = jnp.sum(onehot * t_col, axis=0, keepdims=True)
    out_ref[...] = jnp.concatenate(
        [jnp.transpose(s_col), jnp.transpose(t_col), sw_row, tw_row], axis=0)


_dense_call = pl.pallas_call(
    _dense_body,
    out_shape=jax.ShapeDtypeStruct((4, _NPOS), jnp.float32),
)


def _combine_body(d_ref, ua_ref, up_ref, out_ref):
    d = d_ref[...]
    s = d[0:1, :]
    t = d[1:2, :]
    sw = d[2:3, :]
    tw = d[3:4, :]
    ua = ua_ref[...].reshape(1, _NPOS)
    up = up_ref[...].reshape(1, _NPOS)
    ninv = jnp.float32(1.0 / _NTOT)
    g = jnp.float32(_GAMMA)
    a = (1.0 - g) * ua + g * ninv * sw
    p = (1.0 - g) * up + g * ninv * tw
    out_ref[0, 0] = jnp.sum((p * s - a * t) / (a * a))


_combine_call = pl.pallas_call(
    _combine_body,
    out_shape=jax.ShapeDtypeStruct((1, 1), jnp.float32),
    out_specs=pl.BlockSpec(memory_space=pltpu.SMEM),
)


def kernel(f_ps, f_ns, index_s, u_all, u_pos):
    f_ps = f_ps.reshape(-1).astype(jnp.float32)
    f_ns = f_ns.reshape(-1).astype(jnp.float32)
    idx = index_s.reshape(-1).astype(jnp.int32)

    ua_g, up_g = _sc_gather(idx, u_all, u_pos)
    d = _dense_call(f_ps, f_ns, idx)
    out = _combine_call(d, ua_g.reshape(-1), up_g.reshape(-1))
    return out.reshape(())


# speedup vs baseline: 7.0871x; 7.0871x over previous
import functools

import jax
import jax.numpy as jnp
from jax import lax
from jax.experimental import pallas as pl
from jax.experimental.pallas import tpu as pltpu
from jax.experimental.pallas import tpu_sc as plsc

_GAMMA = 0.9
_NPOS = 1024
_NNEG = 8192
_NTOT = 9216
_CHUNK = 1024


_info = plsc.get_sparse_core_info()
_NC = _info.num_cores
_NS = _info.num_subcores
_NW = _NC * _NS
_BPW = _NPOS // _NW


@functools.partial(
    pl.kernel,
    out_type=[
        jax.ShapeDtypeStruct((_NPOS,), jnp.float32),
        jax.ShapeDtypeStruct((_NPOS,), jnp.float32),
    ],
    mesh=plsc.VectorSubcoreMesh(core_axis_name="c", subcore_axis_name="s"),
    scratch_types=[
        pltpu.VMEM((_BPW,), jnp.int32),
        pltpu.VMEM((_BPW,), jnp.float32),
        pltpu.VMEM((_BPW,), jnp.float32),
        pltpu.SemaphoreType.DMA,
        pltpu.SemaphoreType.DMA,
    ],
)
def _sc_gather(idx_hbm, ua_hbm, up_hbm, oa_hbm, op_hbm,
               idx_v, a_v, p_v, sem_a, sem_p):
    wid = lax.axis_index("s") * _NC + lax.axis_index("c")
    base = wid * _BPW
    pltpu.sync_copy(idx_hbm.at[pl.ds(base, _BPW)], idx_v)
    cp_a = pltpu.async_copy(ua_hbm.at[idx_v], a_v, sem_a)
    cp_p = pltpu.async_copy(up_hbm.at[idx_v], p_v, sem_p)
    cp_a.wait()
    cp_p.wait()
    pltpu.sync_copy(a_v, oa_hbm.at[pl.ds(base, _BPW)])
    pltpu.sync_copy(p_v, op_hbm.at[pl.ds(base, _BPW)])


def _dense_body(fp_ref, fn_ref, idx_ref, out_ref):
    fp_row = fp_ref[...].reshape(1, _NPOS)
    fp_col = jnp.transpose(fp_row)
    a1 = 1.0 - fp_col

    hb = jnp.maximum(a1 + fp_row, 0.0)
    t_col = jnp.sum(hb * hb, axis=1, keepdims=True)
    s_col = t_col
    for c in range(_NNEG // _CHUNK):
        v = fn_ref[...].reshape(1, _NNEG)[:, c * _CHUNK:(c + 1) * _CHUNK]
        hb = jnp.maximum(a1 + v, 0.0)
        s_col = s_col + jnp.sum(hb * hb, axis=1, keepdims=True)

    ir = idx_ref[...].reshape(1, _NPOS)
    ic = jnp.transpose(ir)
    eq = ic == ir
    cid0 = lax.broadcasted_iota(jnp.int32, (_NPOS, _NPOS), 0)
    wmat = jnp.where(eq, cid0, -1)
    w_row = jnp.max(wmat, axis=0, keepdims=True)
    onehot = (wmat == w_row).astype(jnp.float32)

    sw_row = jnp.sum(onehot * s_col, axis=0, keepdims=True)
    tw_row = jnp.sum(onehot * t_col, axis=0, keepdims=True)
    out_ref[...] = jnp.concatenate(
        [jnp.transpose(s_col), jnp.transpose(t_col), sw_row, tw_row], axis=0)


_dense_call = pl.pallas_call(
    _dense_body,
    out_shape=jax.ShapeDtypeStruct((4, _NPOS), jnp.float32),
)


def _combine_body(d_ref, ua_ref, up_ref, out_ref):
    d = d_ref[...]
    s = d[0:1, :]
    t = d[1:2, :]
    sw = d[2:3, :]
    tw = d[3:4, :]
    ua = ua_ref[...].reshape(1, _NPOS)
    up = up_ref[...].reshape(1, _NPOS)
    ninv = jnp.float32(1.0 / _NTOT)
    g = jnp.float32(_GAMMA)
    a = (1.0 - g) * ua + g * ninv * sw
    p = (1.0 - g) * up + g * ninv * tw
    out_ref[0, 0] = jnp.sum((p * s - a * t) / (a * a))


_combine_call = pl.pallas_call(
    _combine_body,
    out_shape=jax.ShapeDtypeStruct((1, 1), jnp.float32),
    out_specs=pl.BlockSpec(memory_space=pltpu.SMEM),
)


def kernel(f_ps, f_ns, index_s, u_all, u_pos):
    f_ps = f_ps.reshape(-1).astype(jnp.float32)
    f_ns = f_ns.reshape(-1).astype(jnp.float32)
    idx = index_s.reshape(-1).astype(jnp.int32)

    ua_g, up_g = _sc_gather(idx, u_all[:, 0], u_pos[:, 0])
    d = _dense_call(f_ps, f_ns, idx)
    out = _combine_call(d, ua_g, up_g)
    return out.reshape(())
